# baseline (speedup 1.0000x reference)
import functools

import jax
import jax.numpy as jnp
from jax import lax
from jax.experimental import pallas as pl
from jax.experimental.pallas import tpu as pltpu
from jax.experimental.pallas import tpu_sc as plsc

_FIELD = 38462
_F = 26
_E = 16
_B = 16384
_NC = 2
_NS = 16
_NW = _NC * _NS
_RPW = _B // _NW
_CHUNK = 128
_NCHUNK = _RPW // _CHUNK


def _sc_body(x_hbm, w_hbm, out_hbm, w_v, x_v, o_v):
    wid = lax.axis_index("s") * _NC + lax.axis_index("c")
    for f in range(_F):
        pltpu.sync_copy(
            w_hbm.at[pl.ds(f * _FIELD, 1), :], w_v.at[pl.ds(f, 1), :]
        )
    base = wid * _RPW
    for c in range(_NCHUNK):
        lo = base + c * _CHUNK
        pltpu.sync_copy(x_hbm.at[pl.ds(lo, _CHUNK), :], x_v)

        def row_body(i, carry):
            bi = jnp.broadcast_to(i, (_E,))
            for f in range(_F):
                bf = jnp.full((_E,), f, jnp.int32)
                xi = plsc.load_gather(x_v, [bi, bf]).astype(jnp.float32)
                o_v[i, pl.ds(f * _E, _E)] = xi * w_v[f]
            return carry

        lax.fori_loop(0, _CHUNK, row_body, 0)
        pltpu.sync_copy(o_v, out_hbm.at[pl.ds(lo, _CHUNK), :])


@jax.jit
def kernel(x, weight):
    mesh = plsc.VectorSubcoreMesh(core_axis_name="c", subcore_axis_name="s")
    run = functools.partial(
        pl.kernel,
        mesh=mesh,
        out_type=jax.ShapeDtypeStruct((_B, _F * _E), jnp.float32),
        scratch_types=[
            pltpu.VMEM((_F, _E), jnp.float32),
            pltpu.VMEM((_CHUNK, _F), jnp.int32),
            pltpu.VMEM((_CHUNK, _F * _E), jnp.float32),
        ],
        compiler_params=pltpu.CompilerParams(needs_layout_passes=False),
    )(_sc_body)
    return run(x, weight)
